# baseline (speedup 1.0000x reference)
import numpy as np
import jax
import jax.numpy as jnp
from jax import lax
from jax.experimental import pallas as pl
from jax.experimental.pallas import tpu as pltpu


def _resize_matrix(in_size: int, out_size: int) -> np.ndarray:
    sample_f = (np.arange(out_size) + 0.5) * (in_size / out_size) - 0.5
    x = np.abs(sample_f[:, None] - np.arange(in_size)[None, :])
    w = np.maximum(0.0, 1.0 - x)
    w = w / w.sum(axis=1, keepdims=True)
    return w.astype(np.float32)


def _index_kernel(lr_doy_ref, hr_doy_ref, tgt_ref,
                  lre_ref, lro_ref, hre_ref, hro_ref):
    t = tgt_ref[...].astype(jnp.float32)

    def one(doy_ref, e_ref, o_ref):
        d = doy_ref[...].astype(jnp.float32)
        T = d.shape[1]
        cmp = (d[:, :, None] < t[0][None, None, :]).astype(jnp.int32)
        idx = jnp.sum(cmp, axis=1)
        i1 = jnp.clip(idx, 1, T - 1)
        i0 = i1 - 1
        even = jnp.where(i0 % 2 == 0, i0, i1)
        e_ref[...] = even
        o_ref[...] = i0 + i1 - even

    one(lr_doy_ref, lre_ref, lro_ref)
    one(hr_doy_ref, hre_ref, hro_ref)


def _fuse_kernel(lre_p, lro_p, hre_p, hro_p, lr_doy_p, hr_doy_p, tgt_p,
                 lre_ref, lro_ref, hre_ref, hro_ref, m_ref,
                 out_lr_ref, out_hr_ref):
    b = pl.program_id(0)
    t = pl.program_id(1)
    tf = tgt_p[t].astype(jnp.float32)

    def weights(doy_p, e_p, o_p):
        e = e_p[b, t]
        o = o_p[b, t]
        i0 = jnp.minimum(e, o)
        i1 = jnp.maximum(e, o)
        d0 = doy_p[b, i0].astype(jnp.float32)
        d1 = doy_p[b, i1].astype(jnp.float32)
        denom = jnp.where(d1 - d0 == 0.0, 1.0, d1 - d0)
        w = jnp.clip((tf - d0) / denom, 0.0, 1.0)
        c = e < o
        return jnp.where(c, 1.0 - w, w), jnp.where(c, w, 1.0 - w)

    whe, who = weights(hr_doy_p, hre_p, hro_p)
    out_hr_ref[0, 0] = hre_ref[0, 0] * whe + hro_ref[0, 0] * who

    wle, wlo = weights(lr_doy_p, lre_p, lro_p)
    lr = lre_ref[0, 0] * wle + lro_ref[0, 0] * wlo
    m = m_ref[...]
    a = lax.dot_general(lr, m, (((1,), (1,)), ((), ())),
                        preferred_element_type=jnp.float32)
    out = lax.dot_general(a, m, (((1,), (1,)), ((), ())),
                          preferred_element_type=jnp.float32)
    out_lr_ref[0, 0] = out


def kernel(lr_data, hr_data, lr_doy, hr_doy, target_doy):
    B, Tl, C, H, W = lr_data.shape
    _, Th, _, Hh, Wh = hr_data.shape
    Tt = target_doy.shape[0]
    Hout, Wout = Hh, Wh
    HWf1 = 256 if (Hh * Wh) % 256 == 0 else Wh
    HWf0 = (Hh * Wh) // HWf1
    hr_flat = hr_data.reshape(B, Th, C, HWf0, HWf1)

    tgt2d = target_doy.reshape(1, Tt)

    lre, lro, hre, hro = pl.pallas_call(
        _index_kernel,
        out_shape=[jax.ShapeDtypeStruct((B, Tt), jnp.int32)] * 4,
    )(lr_doy, hr_doy, tgt2d)

    m = jnp.asarray(_resize_matrix(H, Hout))

    grid_spec = pltpu.PrefetchScalarGridSpec(
        num_scalar_prefetch=7,
        grid=(B, Tt),
        in_specs=[
            pl.BlockSpec((1, 1, C, H, W),
                         lambda b, t, le, lo, he, ho, *_: (b, le[b, t], 0, 0, 0)),
            pl.BlockSpec((1, 1, C, H, W),
                         lambda b, t, le, lo, he, ho, *_: (b, lo[b, t], 0, 0, 0)),
            pl.BlockSpec((1, 1, C, HWf0, HWf1),
                         lambda b, t, le, lo, he, ho, *_: (b, he[b, t], 0, 0, 0)),
            pl.BlockSpec((1, 1, C, HWf0, HWf1),
                         lambda b, t, le, lo, he, ho, *_: (b, ho[b, t], 0, 0, 0)),
            pl.BlockSpec((Hout, H), lambda *_: (0, 0)),
        ],
        out_specs=[
            pl.BlockSpec((1, 1, C, Hout, Wout), lambda b, t, *_: (b, t, 0, 0, 0)),
            pl.BlockSpec((1, 1, C, HWf0, HWf1), lambda b, t, *_: (b, t, 0, 0, 0)),
        ],
    )
    out_lr, out_hr_flat = pl.pallas_call(
        _fuse_kernel,
        grid_spec=grid_spec,
        out_shape=[
            jax.ShapeDtypeStruct((B, Tt, C, Hout, Wout), jnp.float32),
            jax.ShapeDtypeStruct((B, Tt, C, HWf0, HWf1), jnp.float32),
        ],
    )(lre, lro, hre, hro, lr_doy, hr_doy, target_doy,
      lr_data, lr_data, hr_flat, hr_flat, m)

    return (out_lr, out_hr_flat.reshape(B, Tt, C, Hh, Wh))


# speedup vs baseline: 2.2549x; 2.2549x over previous
import numpy as np
import jax
import jax.numpy as jnp
from jax import lax
from jax.experimental import pallas as pl
from jax.experimental.pallas import tpu as pltpu


def _resize_matrix(in_size: int, out_size: int) -> np.ndarray:
    sample_f = (np.arange(out_size) + 0.5) * (in_size / out_size) - 0.5
    x = np.abs(sample_f[:, None] - np.arange(in_size)[None, :])
    w = np.maximum(0.0, 1.0 - x)
    w = w / w.sum(axis=1, keepdims=True)
    return w.astype(np.float32)


def _index_kernel(lr_doy_ref, hr_doy_ref, tgt_ref,
                  lre_ref, lro_ref, hre_ref, hro_ref):
    t = tgt_ref[...].astype(jnp.float32)

    def one(doy_ref, e_ref, o_ref):
        d = doy_ref[...].astype(jnp.float32)
        T = d.shape[1]
        cmp = (d[:, :, None] < t[0][None, None, :]).astype(jnp.int32)
        idx = jnp.sum(cmp, axis=1)
        i1 = jnp.clip(idx, 1, T - 1)
        i0 = i1 - 1
        even = jnp.where(i0 % 2 == 0, i0, i1)
        e_ref[...] = even
        o_ref[...] = i0 + i1 - even

    one(lr_doy_ref, lre_ref, lro_ref)
    one(hr_doy_ref, hre_ref, hro_ref)


def _fuse_kernel(lre_p, lro_p, hre_p, hro_p, lr_doy_p, hr_doy_p, tgt_p,
                 lre_ref, lro_ref, hre_ref, hro_ref, m_ref,
                 out_lr_ref, out_hr_ref):
    b = pl.program_id(0)
    t = pl.program_id(1)
    tf = tgt_p[t].astype(jnp.float32)

    def weights(doy_p, e_p, o_p):
        e = e_p[b, t]
        o = o_p[b, t]
        i0 = jnp.minimum(e, o)
        i1 = jnp.maximum(e, o)
        d0 = doy_p[b, i0].astype(jnp.float32)
        d1 = doy_p[b, i1].astype(jnp.float32)
        denom = jnp.where(d1 - d0 == 0.0, 1.0, d1 - d0)
        w = jnp.clip((tf - d0) / denom, 0.0, 1.0)
        c = e < o
        return jnp.where(c, 1.0 - w, w), jnp.where(c, w, 1.0 - w)

    whe, who = weights(hr_doy_p, hre_p, hro_p)
    out_hr_ref[0, 0] = hre_ref[0, 0] * whe + hro_ref[0, 0] * who

    wle, wlo = weights(lr_doy_p, lre_p, lro_p)
    lr = lre_ref[0, 0] * wle + lro_ref[0, 0] * wlo
    m = m_ref[...]
    a = lax.dot_general(lr, m, (((1,), (1,)), ((), ())),
                        preferred_element_type=jnp.float32)
    out = lax.dot_general(a, m, (((1,), (1,)), ((), ())),
                          preferred_element_type=jnp.float32)
    out_lr_ref[0, 0] = out


def kernel(lr_data, hr_data, lr_doy, hr_doy, target_doy):
    B, Tl, C, H, W = lr_data.shape
    _, Th, _, Hh, Wh = hr_data.shape
    Tt = target_doy.shape[0]
    Hout, Wout = Hh, Wh
    tgt2d = target_doy.reshape(1, Tt)

    lre, lro, hre, hro = pl.pallas_call(
        _index_kernel,
        out_shape=[jax.ShapeDtypeStruct((B, Tt), jnp.int32)] * 4,
    )(lr_doy, hr_doy, tgt2d)

    m = jnp.asarray(_resize_matrix(H, Hout))

    grid_spec = pltpu.PrefetchScalarGridSpec(
        num_scalar_prefetch=7,
        grid=(B, Tt),
        in_specs=[
            pl.BlockSpec((1, 1, C, H, W),
                         lambda b, t, le, lo, he, ho, *_: (b, le[b, t], 0, 0, 0)),
            pl.BlockSpec((1, 1, C, H, W),
                         lambda b, t, le, lo, he, ho, *_: (b, lo[b, t], 0, 0, 0)),
            pl.BlockSpec((1, 1, C, Hh, Wh),
                         lambda b, t, le, lo, he, ho, *_: (b, he[b, t], 0, 0, 0)),
            pl.BlockSpec((1, 1, C, Hh, Wh),
                         lambda b, t, le, lo, he, ho, *_: (b, ho[b, t], 0, 0, 0)),
            pl.BlockSpec((Hout, H), lambda *_: (0, 0)),
        ],
        out_specs=[
            pl.BlockSpec((1, 1, C, Hout, Wout), lambda b, t, *_: (b, t, 0, 0, 0)),
            pl.BlockSpec((1, 1, C, Hh, Wh), lambda b, t, *_: (b, t, 0, 0, 0)),
        ],
    )
    out_lr, out_hr = pl.pallas_call(
        _fuse_kernel,
        grid_spec=grid_spec,
        out_shape=[
            jax.ShapeDtypeStruct((B, Tt, C, Hout, Wout), jnp.float32),
            jax.ShapeDtypeStruct((B, Tt, C, Hh, Wh), jnp.float32),
        ],
    )(lre, lro, hre, hro, lr_doy, hr_doy, target_doy,
      lr_data, lr_data, hr_data, hr_data, m)

    return (out_lr, out_hr)
